# baseline (speedup 1.0000x reference)
import jax
import jax.numpy as jnp
from jax import lax
from jax.experimental import pallas as pl
from jax.experimental.pallas import tpu as pltpu
from jax.experimental.pallas import tpu_sc as plsc

_N = 1000000
_B = 16384
_F = 64
_NW = 32
_BPW = _B // _NW
_G = 16
_GROUPS = _BPW // _G
_CHUNK = 128
_NCHUNK = _BPW // _CHUNK


def _user_body(user_hbm, utab_hbm, eut_hbm,
               uidx_v, ubuf_v, eut_v, sem):
    nc = 2
    wid = lax.axis_index("s") * nc + lax.axis_index("c")
    base = wid * _BPW

    pltpu.sync_copy(user_hbm.at[pl.ds(base, _BPW)], uidx_v)
    lane = lax.iota(jnp.int32, 16)

    def fire_group(g):
        buf = lax.rem(g, 2)
        urows = uidx_v[pl.ds(g * _G, _G)]
        for j in range(_G):
            ub = pl.multiple_of(jnp.bitwise_and(urows[j], -8), 8)
            pltpu.async_copy(utab_hbm.at[pl.ds(ub, 8), :],
                             ubuf_v.at[buf, pl.ds(j * 8, 8), :], sem)

    def drain_group(g):
        buf = lax.rem(g, 2)
        for j in range(_G):
            pltpu.make_async_copy(utab_hbm.at[pl.ds(0, 8), :],
                                  ubuf_v.at[buf, pl.ds(j * 8, 8), :], sem).wait()

    def extract_group(g):
        buf = lax.rem(g, 2)
        goff = g * _G
        usub = jnp.bitwise_and(uidx_v[pl.ds(goff, _G)], 7) + lane * 8
        for f in range(_F):
            colf = jnp.full((16,), f, jnp.int32)
            eut_v[f, pl.ds(goff, _G)] = plsc.load_gather(
                ubuf_v.at[buf], [usub, colf])

    fire_group(0)

    def group_body(g, carry):
        fire_group(g + 1)
        drain_group(g)
        extract_group(g)
        return carry

    lax.fori_loop(0, _GROUPS - 1, group_body, 0, unroll=False)
    drain_group(_GROUPS - 1)
    extract_group(_GROUPS - 1)

    pltpu.sync_copy(eut_v, eut_hbm.at[:, pl.ds(base, _BPW)])


def _item_body(item_hbm, itab_hbm, eut_hbm, wb_hbm, out_hbm,
               iidx_v, eut_v, irows_v, wb_v, out_v, sem, sem2):
    nc = 2
    wid = lax.axis_index("s") * nc + lax.axis_index("c")
    base = wid * _BPW

    pltpu.sync_copy(item_hbm.at[pl.ds(wid * _NCHUNK, _NCHUNK)], iidx_v)
    pltpu.sync_copy(wb_hbm, wb_v)
    eut_desc = pltpu.make_async_copy(eut_hbm.at[:, pl.ds(base, _BPW)],
                                     eut_v, sem2)
    eut_desc.start()

    descs = []
    for k in range(_NCHUNK):
        descs.append(pltpu.async_copy(
            itab_hbm.at[iidx_v.at[k]],
            irows_v.at[pl.ds(k * _CHUNK, _CHUNK)], sem))
    for d in descs:
        d.wait()
    eut_desc.wait()

    wvecs = [wb_v[pl.ds(c * 16, 16)] for c in range(_F // 16)]
    bvec = wb_v[pl.ds(_F, 16)]
    lane = lax.iota(jnp.int32, 16)

    def group_body(g, carry):
        goff = g * _G
        rows = goff + lane
        acc = bvec
        for f in range(_F):
            wf = wvecs[f // 16][f % 16]
            colf = jnp.full((16,), f, jnp.int32)
            iv = plsc.load_gather(irows_v, [rows, colf])
            u = eut_v[f, pl.ds(goff, _G)]
            acc = acc + u * iv * wf
        out_v[pl.ds(goff, _G)] = acc
        return carry

    lax.fori_loop(0, _GROUPS, group_body, 0, unroll=False)

    pltpu.sync_copy(out_v, out_hbm.at[pl.ds(base, _BPW)])


def kernel(user, item, embed_user_w, embed_item_w, predict_w, predict_b):
    wb = jnp.concatenate([predict_w.reshape(_F).astype(jnp.float32),
                          jnp.broadcast_to(predict_b.astype(jnp.float32), (16,))])

    mesh = plsc.VectorSubcoreMesh(core_axis_name="c", subcore_axis_name="s")
    run_user = pl.kernel(
        _user_body,
        out_type=jax.ShapeDtypeStruct((_F, _B), jnp.float32),
        mesh=mesh,
        compiler_params=pltpu.CompilerParams(needs_layout_passes=False,
                                             use_tc_tiling_on_sc=True),
        scratch_types=[
            pltpu.VMEM((_BPW,), jnp.int32),
            pltpu.VMEM((2, _G * 8, _F), jnp.float32),
            pltpu.VMEM((_F, _BPW), jnp.float32),
            pltpu.SemaphoreType.DMA,
        ],
    )
    eut = run_user(user.astype(jnp.int32), embed_user_w)

    run_item = pl.kernel(
        _item_body,
        out_type=jax.ShapeDtypeStruct((_B,), jnp.float32),
        mesh=mesh,
        compiler_params=pltpu.CompilerParams(needs_layout_passes=False,
                                             use_tc_tiling_on_sc=False),
        scratch_types=[
            pltpu.VMEM((_NCHUNK, _CHUNK), jnp.int32),
            pltpu.VMEM((_F, _BPW), jnp.float32),
            pltpu.VMEM((_BPW, _F), jnp.float32),
            pltpu.VMEM((_F + 16,), jnp.float32),
            pltpu.VMEM((_BPW,), jnp.float32),
            pltpu.SemaphoreType.DMA,
            pltpu.SemaphoreType.DMA,
        ],
    )
    item2d = item.astype(jnp.int32).reshape(_B // _CHUNK, _CHUNK)
    return run_item(item2d, embed_item_w, eut, wb)
